# Initial kernel scaffold; baseline (speedup 1.0000x reference)
import jax
import jax.numpy as jnp
from jax.experimental import pallas as pl


def kernel(node_feat, edge_index, W1, b1, W2, b2):
    raise NotImplementedError("write your pallas kernel here")


# speedup vs baseline: 3.4911x; 3.4911x over previous
import jax
import jax.numpy as jnp
from jax import lax
from jax.experimental import pallas as pl
from jax.experimental.pallas import tpu as pltpu
from jax.experimental.pallas import tpu_sc as plsc

N = 10000
E = 160000
IN_F = 256
HID_F = 512
OUT_F = 256
HALF = 128
NC = 2
NS = 16
K = 128
CH = 80
EPT = CH * K
EP = NS * EPT
NPAD = 10112
ACC_PT = NPAD // NS
OUT_PT = 624
BM = 1000

_MESH = plsc.VectorSubcoreMesh(core_axis_name="c", subcore_axis_name="s")


def _agg_body(feat, srcs, dsts, zeros, msg, src_v, dst_v, rows_v, acc, sem):
  c = lax.axis_index("c")
  s = lax.axis_index("s")

  pltpu.sync_copy(srcs.at[c, s], src_v)
  pltpu.sync_copy(dsts.at[s], dst_v)
  pltpu.sync_copy(zeros, acc.at[pl.ds(s * ACC_PT, ACC_PT)])
  plsc.subcore_barrier()

  def chunk(j, carry):
    pltpu.async_copy(feat.at[src_v.at[j]], rows_v, sem).wait()
    pltpu.sync_copy(rows_v, acc.at[dst_v.at[j]], add=True)
    return carry

  lax.fori_loop(0, CH, chunk, 0)
  plsc.subcore_barrier()

  pltpu.sync_copy(acc.at[pl.ds(s * OUT_PT, OUT_PT)],
                  msg.at[c, pl.ds(s * OUT_PT, OUT_PT)])

  @pl.when(s == NS - 1)
  def _():
    pltpu.sync_copy(acc.at[pl.ds(NS * OUT_PT, N - NS * OUT_PT)],
                    msg.at[c, pl.ds(NS * OUT_PT, N - NS * OUT_PT)])


_agg = pl.kernel(
    _agg_body,
    out_type=jax.ShapeDtypeStruct((NC, N, HALF), jnp.float32),
    mesh=_MESH,
    scratch_types=[
        pltpu.VMEM((CH, K), jnp.int32),
        pltpu.VMEM((CH, K), jnp.int32),
        pltpu.VMEM((K, HALF), jnp.float32),
        pltpu.VMEM_SHARED((NPAD, HALF), jnp.float32),
        pltpu.SemaphoreType.DMA,
    ],
)


def _deg_body(dsts, zeros, ones, degw, dst_v, ones_v, dacc):
  c = lax.axis_index("c")
  s = lax.axis_index("s")

  pltpu.sync_copy(dsts.at[s], dst_v)
  pltpu.sync_copy(ones, ones_v)
  pltpu.sync_copy(zeros, dacc.at[pl.ds(s * ACC_PT, ACC_PT)])
  plsc.subcore_barrier()

  def chunk(j, carry):
    pltpu.sync_copy(ones_v, dacc.at[dst_v.at[j]], add=True)
    return carry

  lax.fori_loop(c * (CH // 2), (c + 1) * (CH // 2), chunk, 0)
  plsc.subcore_barrier()

  pltpu.sync_copy(dacc.at[pl.ds(s * OUT_PT, OUT_PT)],
                  degw.at[c, pl.ds(s * OUT_PT, OUT_PT)])

  @pl.when(s == NS - 1)
  def _():
    pltpu.sync_copy(dacc.at[pl.ds(NS * OUT_PT, N - NS * OUT_PT)],
                    degw.at[c, pl.ds(NS * OUT_PT, N - NS * OUT_PT)])


_deg = pl.kernel(
    _deg_body,
    out_type=jax.ShapeDtypeStruct((NC, N, HALF), jnp.float32),
    mesh=_MESH,
    scratch_types=[
        pltpu.VMEM((CH, K), jnp.int32),
        pltpu.VMEM((K, HALF), jnp.float32),
        pltpu.VMEM_SHARED((NPAD, HALF), jnp.float32),
    ],
)


def _relayout_body(x_ref, o_ref):
  o_ref[0] = x_ref[:, :HALF]
  o_ref[1] = x_ref[:, HALF:]


_relayout = pl.pallas_call(
    _relayout_body,
    grid=(N // BM,),
    in_specs=[pl.BlockSpec((BM, IN_F), lambda i: (i, 0))],
    out_specs=pl.BlockSpec((NC, BM, HALF), lambda i: (0, i, 0)),
    out_shape=jax.ShapeDtypeStruct((NC, N, HALF), jnp.float32),
)


def _layer1_body(msg_ref, h_ref, degw_ref, w1_ref, b1_ref, w2_ref, g_ref):
  m = jnp.concatenate([msg_ref[0], msg_ref[1]], axis=1)
  inv = 1.0 / (degw_ref[0, :, 0:1] + degw_ref[1, :, 0:1] + 1.0)
  hn = (m + h_ref[...]) * inv
  h1 = jnp.dot(hn, w1_ref[...], preferred_element_type=jnp.float32,
               precision=lax.Precision.HIGHEST) + b1_ref[...]
  h1 = jnp.maximum(h1, 0.0)
  g = jnp.dot(h1, w2_ref[...], preferred_element_type=jnp.float32,
              precision=lax.Precision.HIGHEST)
  g_ref[0] = g[:, :HALF]
  g_ref[1] = g[:, HALF:]


_layer1 = pl.pallas_call(
    _layer1_body,
    grid=(N // BM,),
    in_specs=[
        pl.BlockSpec((NC, BM, HALF), lambda i: (0, i, 0)),
        pl.BlockSpec((BM, IN_F), lambda i: (i, 0)),
        pl.BlockSpec((NC, BM, HALF), lambda i: (0, i, 0)),
        pl.BlockSpec((IN_F, HID_F), lambda i: (0, 0)),
        pl.BlockSpec((1, HID_F), lambda i: (0, 0)),
        pl.BlockSpec((HID_F, OUT_F), lambda i: (0, 0)),
    ],
    out_specs=pl.BlockSpec((NC, BM, HALF), lambda i: (0, i, 0)),
    out_shape=jax.ShapeDtypeStruct((NC, N, HALF), jnp.float32),
)


def _layer2_body(msg_ref, g_ref, degw_ref, b2_ref, o_ref):
  m = jnp.concatenate([msg_ref[0], msg_ref[1]], axis=1)
  g = jnp.concatenate([g_ref[0], g_ref[1]], axis=1)
  inv = 1.0 / (degw_ref[0, :, 0:1] + degw_ref[1, :, 0:1] + 1.0)
  o_ref[...] = (m + g) * inv + b2_ref[...]


_layer2 = pl.pallas_call(
    _layer2_body,
    grid=(N // BM,),
    in_specs=[
        pl.BlockSpec((NC, BM, HALF), lambda i: (0, i, 0)),
        pl.BlockSpec((NC, BM, HALF), lambda i: (0, i, 0)),
        pl.BlockSpec((NC, BM, HALF), lambda i: (0, i, 0)),
        pl.BlockSpec((1, OUT_F), lambda i: (0, 0)),
    ],
    out_specs=pl.BlockSpec((BM, OUT_F), lambda i: (i, 0)),
    out_shape=jax.ShapeDtypeStruct((N, OUT_F), jnp.float32),
)


def kernel(node_feat, edge_index, W1, b1, W2, b2):
  src = edge_index[0]
  dst = edge_index[1]
  pad = EP - E
  src_p = jnp.concatenate([src, jnp.zeros((pad,), jnp.int32)])
  dst_p = jnp.concatenate([dst, jnp.full((pad,), N, jnp.int32)])
  srcs = jnp.stack([src_p, src_p + N]).reshape(NC, NS, CH, K)
  dsts = dst_p.reshape(NS, CH, K)
  zeros = jnp.zeros((ACC_PT, HALF), jnp.float32)
  ones = jnp.ones((K, HALF), jnp.float32)

  feat_t = _relayout(node_feat)
  degw = _deg(dsts, zeros, ones)
  msg1 = _agg(feat_t.reshape(NC * N, HALF), srcs, dsts, zeros)
  g = _layer1(msg1, node_feat, degw, W1, b1.reshape(1, HID_F), W2)
  msg2 = _agg(g.reshape(NC * N, HALF), srcs, dsts, zeros)
  return _layer2(msg2, g, degw, b2.reshape(1, OUT_F))


# speedup vs baseline: 3.8848x; 1.1128x over previous
import jax
import jax.numpy as jnp
from jax import lax
from jax.experimental import pallas as pl
from jax.experimental.pallas import tpu as pltpu
from jax.experimental.pallas import tpu_sc as plsc

N = 10000
E = 160000
IN_F = 256
HID_F = 512
OUT_F = 256
HALF = 128
NC = 2
NS = 16
K = 128
CH = 80
CHH = 40
EPT = CH * K
EP = NS * EPT
NPAD = 10112
ACC_PT = NPAD // NS
OUT_PT = 624
BM = 1000

_MESH = plsc.VectorSubcoreMesh(core_axis_name="c", subcore_axis_name="s")


def _agg_body(feat, srcs, dsts, zeros, msg,
              src_v, dst_v, rows_a, rows_b, acc, ga, gb, sa, sb):
  c = lax.axis_index("c")
  s = lax.axis_index("s")

  pltpu.sync_copy(zeros, acc.at[pl.ds(s * ACC_PT, ACC_PT)])
  plsc.subcore_barrier()

  def gather(j, buf, sem):
    pltpu.async_copy(feat.at[src_v.at[j]], buf, sem)

  def wait_gather(j, buf, sem):
    pltpu.make_async_copy(feat.at[src_v.at[j]], buf, sem).wait()

  def scatter(j, buf, sem):
    pltpu.async_copy(buf, acc.at[dst_v.at[j]], sem, add=True)

  def wait_scatter(j, buf, sem):
    pltpu.make_async_copy(buf, acc.at[dst_v.at[j]], sem).wait()

  for st in range(CH // CHH):
    pltpu.sync_copy(srcs.at[c, s, pl.ds(st * CHH, CHH)], src_v)
    pltpu.sync_copy(dsts.at[s, pl.ds(st * CHH, CHH)], dst_v)
    gather(0, rows_a, ga)
    gather(1, rows_b, gb)

    def pair(p, carry):
      j0 = 2 * p
      j1 = 2 * p + 1
      wait_gather(j0, rows_a, ga)
      scatter(j0, rows_a, sa)
      wait_gather(j1, rows_b, gb)
      scatter(j1, rows_b, sb)
      wait_scatter(j0, rows_a, sa)
      gather(j0 + 2, rows_a, ga)
      wait_scatter(j1, rows_b, sb)
      gather(j1 + 2, rows_b, gb)
      return carry

    lax.fori_loop(0, CHH // 2 - 1, pair, 0)
    j0 = CHH - 2
    j1 = CHH - 1
    wait_gather(j0, rows_a, ga)
    scatter(j0, rows_a, sa)
    wait_gather(j1, rows_b, gb)
    scatter(j1, rows_b, sb)
    wait_scatter(j0, rows_a, sa)
    wait_scatter(j1, rows_b, sb)

  plsc.subcore_barrier()

  pltpu.sync_copy(acc.at[pl.ds(s * OUT_PT, OUT_PT)],
                  msg.at[c, pl.ds(s * OUT_PT, OUT_PT)])

  @pl.when(s == NS - 1)
  def _():
    pltpu.sync_copy(acc.at[pl.ds(NS * OUT_PT, N - NS * OUT_PT)],
                    msg.at[c, pl.ds(NS * OUT_PT, N - NS * OUT_PT)])


_agg = pl.kernel(
    _agg_body,
    out_type=jax.ShapeDtypeStruct((NC, N, HALF), jnp.float32),
    mesh=_MESH,
    scratch_types=[
        pltpu.VMEM((CHH, K), jnp.int32),
        pltpu.VMEM((CHH, K), jnp.int32),
        pltpu.VMEM((K, HALF), jnp.float32),
        pltpu.VMEM((K, HALF), jnp.float32),
        pltpu.VMEM_SHARED((NPAD, HALF), jnp.float32),
        pltpu.SemaphoreType.DMA,
        pltpu.SemaphoreType.DMA,
        pltpu.SemaphoreType.DMA,
        pltpu.SemaphoreType.DMA,
    ],
)


def _deg_body(dsts, zeros, ones, degw, dst_v, ones_v, dacc):
  c = lax.axis_index("c")
  s = lax.axis_index("s")

  pltpu.sync_copy(dsts.at[s], dst_v)
  pltpu.sync_copy(ones, ones_v)
  pltpu.sync_copy(zeros, dacc.at[pl.ds(s * ACC_PT, ACC_PT)])
  plsc.subcore_barrier()

  def chunk(j, carry):
    pltpu.sync_copy(ones_v, dacc.at[dst_v.at[j]], add=True)
    return carry

  lax.fori_loop(c * (CH // 2), (c + 1) * (CH // 2), chunk, 0)
  plsc.subcore_barrier()

  pltpu.sync_copy(dacc.at[pl.ds(s * OUT_PT, OUT_PT)],
                  degw.at[c, pl.ds(s * OUT_PT, OUT_PT)])

  @pl.when(s == NS - 1)
  def _():
    pltpu.sync_copy(dacc.at[pl.ds(NS * OUT_PT, N - NS * OUT_PT)],
                    degw.at[c, pl.ds(NS * OUT_PT, N - NS * OUT_PT)])


_deg = pl.kernel(
    _deg_body,
    out_type=jax.ShapeDtypeStruct((NC, N, HALF), jnp.float32),
    mesh=_MESH,
    scratch_types=[
        pltpu.VMEM((CH, K), jnp.int32),
        pltpu.VMEM((K, HALF), jnp.float32),
        pltpu.VMEM_SHARED((NPAD, HALF), jnp.float32),
    ],
)


def _relayout_body(x_ref, o_ref):
  o_ref[0] = x_ref[:, :HALF]
  o_ref[1] = x_ref[:, HALF:]


_relayout = pl.pallas_call(
    _relayout_body,
    grid=(N // BM,),
    in_specs=[pl.BlockSpec((BM, IN_F), lambda i: (i, 0))],
    out_specs=pl.BlockSpec((NC, BM, HALF), lambda i: (0, i, 0)),
    out_shape=jax.ShapeDtypeStruct((NC, N, HALF), jnp.float32),
)


def _layer1_body(msg_ref, h_ref, degw_ref, w1_ref, b1_ref, w2_ref, g_ref):
  m = jnp.concatenate([msg_ref[0], msg_ref[1]], axis=1)
  inv = 1.0 / (degw_ref[0, :, 0:1] + degw_ref[1, :, 0:1] + 1.0)
  hn = (m + h_ref[...]) * inv
  h1 = jnp.dot(hn, w1_ref[...], preferred_element_type=jnp.float32,
               precision=lax.Precision.HIGHEST) + b1_ref[...]
  h1 = jnp.maximum(h1, 0.0)
  g = jnp.dot(h1, w2_ref[...], preferred_element_type=jnp.float32,
              precision=lax.Precision.HIGHEST)
  g_ref[0] = g[:, :HALF]
  g_ref[1] = g[:, HALF:]


_layer1 = pl.pallas_call(
    _layer1_body,
    grid=(N // BM,),
    in_specs=[
        pl.BlockSpec((NC, BM, HALF), lambda i: (0, i, 0)),
        pl.BlockSpec((BM, IN_F), lambda i: (i, 0)),
        pl.BlockSpec((NC, BM, HALF), lambda i: (0, i, 0)),
        pl.BlockSpec((IN_F, HID_F), lambda i: (0, 0)),
        pl.BlockSpec((1, HID_F), lambda i: (0, 0)),
        pl.BlockSpec((HID_F, OUT_F), lambda i: (0, 0)),
    ],
    out_specs=pl.BlockSpec((NC, BM, HALF), lambda i: (0, i, 0)),
    out_shape=jax.ShapeDtypeStruct((NC, N, HALF), jnp.float32),
)


def _layer2_body(msg_ref, g_ref, degw_ref, b2_ref, o_ref):
  m = jnp.concatenate([msg_ref[0], msg_ref[1]], axis=1)
  g = jnp.concatenate([g_ref[0], g_ref[1]], axis=1)
  inv = 1.0 / (degw_ref[0, :, 0:1] + degw_ref[1, :, 0:1] + 1.0)
  o_ref[...] = (m + g) * inv + b2_ref[...]


_layer2 = pl.pallas_call(
    _layer2_body,
    grid=(N // BM,),
    in_specs=[
        pl.BlockSpec((NC, BM, HALF), lambda i: (0, i, 0)),
        pl.BlockSpec((NC, BM, HALF), lambda i: (0, i, 0)),
        pl.BlockSpec((NC, BM, HALF), lambda i: (0, i, 0)),
        pl.BlockSpec((1, OUT_F), lambda i: (0, 0)),
    ],
    out_specs=pl.BlockSpec((BM, OUT_F), lambda i: (i, 0)),
    out_shape=jax.ShapeDtypeStruct((N, OUT_F), jnp.float32),
)


def kernel(node_feat, edge_index, W1, b1, W2, b2):
  src = edge_index[0]
  dst = edge_index[1]
  pad = EP - E
  src_p = jnp.concatenate([src, jnp.zeros((pad,), jnp.int32)])
  dst_p = jnp.concatenate([dst, jnp.full((pad,), N, jnp.int32)])
  srcs = jnp.stack([src_p, src_p + N]).reshape(NC, NS, CH, K)
  dsts = dst_p.reshape(NS, CH, K)
  zeros = jnp.zeros((ACC_PT, HALF), jnp.float32)
  ones = jnp.ones((K, HALF), jnp.float32)

  feat_t = _relayout(node_feat)
  degw = _deg(dsts, zeros, ones)
  msg1 = _agg(feat_t.reshape(NC * N, HALF), srcs, dsts, zeros)
  g = _layer1(msg1, node_feat, degw, W1, b1.reshape(1, HID_F), W2)
  msg2 = _agg(g.reshape(NC * N, HALF), srcs, dsts, zeros)
  return _layer2(msg2, g, degw, b2.reshape(1, OUT_F))


# speedup vs baseline: 3.9469x; 1.0160x over previous
import jax
import jax.numpy as jnp
from jax import lax
from jax.experimental import pallas as pl
from jax.experimental.pallas import tpu as pltpu
from jax.experimental.pallas import tpu_sc as plsc

N = 10000
E = 160000
IN_F = 256
HID_F = 512
OUT_F = 256
HALF = 128
NC = 2
NS = 16
K = 128
CH = 80
CHH = 40
EPT = CH * K
EP = NS * EPT
NPAD = 10112
ACC_PT = NPAD // NS
OUT_PT = 624
BM = 1000

_MESH = plsc.VectorSubcoreMesh(core_axis_name="c", subcore_axis_name="s")


def _agg_body(feat, srcs, dsts, zeros, msg,
              src_v, dst_v, rows_a, rows_b, acc, ga, gb, sa, sb):
  c = lax.axis_index("c")
  s = lax.axis_index("s")

  pltpu.sync_copy(zeros, acc.at[pl.ds(s * ACC_PT, ACC_PT)])
  plsc.subcore_barrier()

  def gather(j, buf, sem):
    pltpu.async_copy(feat.at[src_v.at[j]], buf, sem)

  def wait_gather(j, buf, sem):
    pltpu.make_async_copy(feat.at[src_v.at[j]], buf, sem).wait()

  def scatter(j, buf, sem):
    pltpu.async_copy(buf, acc.at[dst_v.at[j]], sem, add=True)

  def wait_scatter(j, buf, sem):
    pltpu.make_async_copy(buf, acc.at[dst_v.at[j]], sem).wait()

  for st in range(CH // CHH):
    pltpu.sync_copy(srcs.at[c, s, pl.ds(st * CHH, CHH)], src_v)
    pltpu.sync_copy(dsts.at[s, pl.ds(st * CHH, CHH)], dst_v)
    gather(0, rows_a, ga)

    def pair(i, carry):
      j0 = 2 * i
      j1 = 2 * i + 1
      wait_gather(j0, rows_a, ga)
      scatter(j0, rows_a, sa)

      @pl.when(i > 0)
      def _():
        wait_scatter(j1 - 2, rows_b, sb)

      gather(j1, rows_b, gb)
      wait_gather(j1, rows_b, gb)
      scatter(j1, rows_b, sb)
      wait_scatter(j0, rows_a, sa)

      @pl.when(i < CHH // 2 - 1)
      def _():
        gather(j0 + 2, rows_a, ga)

      return carry

    lax.fori_loop(0, CHH // 2, pair, 0)
    wait_scatter(CHH - 1, rows_b, sb)

  plsc.subcore_barrier()

  pltpu.sync_copy(acc.at[pl.ds(s * OUT_PT, OUT_PT)],
                  msg.at[c, pl.ds(s * OUT_PT, OUT_PT)])

  @pl.when(s == NS - 1)
  def _():
    pltpu.sync_copy(acc.at[pl.ds(NS * OUT_PT, N - NS * OUT_PT)],
                    msg.at[c, pl.ds(NS * OUT_PT, N - NS * OUT_PT)])


_agg = pl.kernel(
    _agg_body,
    out_type=jax.ShapeDtypeStruct((NC, N, HALF), jnp.float32),
    mesh=_MESH,
    scratch_types=[
        pltpu.VMEM((CHH, K), jnp.int32),
        pltpu.VMEM((CHH, K), jnp.int32),
        pltpu.VMEM((K, HALF), jnp.float32),
        pltpu.VMEM((K, HALF), jnp.float32),
        pltpu.VMEM_SHARED((NPAD, HALF), jnp.float32),
        pltpu.SemaphoreType.DMA,
        pltpu.SemaphoreType.DMA,
        pltpu.SemaphoreType.DMA,
        pltpu.SemaphoreType.DMA,
    ],
)


def _deg_body(dsts, zeros, ones, degw, dst_v, ones_v, dacc):
  c = lax.axis_index("c")
  s = lax.axis_index("s")

  pltpu.sync_copy(dsts.at[s], dst_v)
  pltpu.sync_copy(ones, ones_v)
  pltpu.sync_copy(zeros, dacc.at[pl.ds(s * ACC_PT, ACC_PT)])
  plsc.subcore_barrier()

  def chunk(j, carry):
    pltpu.sync_copy(ones_v, dacc.at[dst_v.at[j]], add=True)
    return carry

  lax.fori_loop(c * (CH // 2), (c + 1) * (CH // 2), chunk, 0)
  plsc.subcore_barrier()

  pltpu.sync_copy(dacc.at[pl.ds(s * OUT_PT, OUT_PT)],
                  degw.at[c, pl.ds(s * OUT_PT, OUT_PT)])

  @pl.when(s == NS - 1)
  def _():
    pltpu.sync_copy(dacc.at[pl.ds(NS * OUT_PT, N - NS * OUT_PT)],
                    degw.at[c, pl.ds(NS * OUT_PT, N - NS * OUT_PT)])


_deg = pl.kernel(
    _deg_body,
    out_type=jax.ShapeDtypeStruct((NC, N, HALF), jnp.float32),
    mesh=_MESH,
    scratch_types=[
        pltpu.VMEM((CH, K), jnp.int32),
        pltpu.VMEM((K, HALF), jnp.float32),
        pltpu.VMEM_SHARED((NPAD, HALF), jnp.float32),
    ],
)


def _relayout_body(x_ref, o_ref):
  o_ref[0] = x_ref[:, :HALF]
  o_ref[1] = x_ref[:, HALF:]


_relayout = pl.pallas_call(
    _relayout_body,
    grid=(N // BM,),
    in_specs=[pl.BlockSpec((BM, IN_F), lambda i: (i, 0))],
    out_specs=pl.BlockSpec((NC, BM, HALF), lambda i: (0, i, 0)),
    out_shape=jax.ShapeDtypeStruct((NC, N, HALF), jnp.float32),
)


def _layer1_body(msg_ref, h_ref, degw_ref, w1_ref, b1_ref, w2_ref, g_ref):
  m = jnp.concatenate([msg_ref[0], msg_ref[1]], axis=1)
  inv = 1.0 / (degw_ref[0, :, 0:1] + degw_ref[1, :, 0:1] + 1.0)
  hn = (m + h_ref[...]) * inv
  h1 = jnp.dot(hn, w1_ref[...], preferred_element_type=jnp.float32,
               precision=lax.Precision.HIGHEST) + b1_ref[...]
  h1 = jnp.maximum(h1, 0.0)
  g = jnp.dot(h1, w2_ref[...], preferred_element_type=jnp.float32,
              precision=lax.Precision.HIGHEST)
  g_ref[0] = g[:, :HALF]
  g_ref[1] = g[:, HALF:]


_layer1 = pl.pallas_call(
    _layer1_body,
    grid=(N // BM,),
    in_specs=[
        pl.BlockSpec((NC, BM, HALF), lambda i: (0, i, 0)),
        pl.BlockSpec((BM, IN_F), lambda i: (i, 0)),
        pl.BlockSpec((NC, BM, HALF), lambda i: (0, i, 0)),
        pl.BlockSpec((IN_F, HID_F), lambda i: (0, 0)),
        pl.BlockSpec((1, HID_F), lambda i: (0, 0)),
        pl.BlockSpec((HID_F, OUT_F), lambda i: (0, 0)),
    ],
    out_specs=pl.BlockSpec((NC, BM, HALF), lambda i: (0, i, 0)),
    out_shape=jax.ShapeDtypeStruct((NC, N, HALF), jnp.float32),
)


def _layer2_body(msg_ref, g_ref, degw_ref, b2_ref, o_ref):
  m = jnp.concatenate([msg_ref[0], msg_ref[1]], axis=1)
  g = jnp.concatenate([g_ref[0], g_ref[1]], axis=1)
  inv = 1.0 / (degw_ref[0, :, 0:1] + degw_ref[1, :, 0:1] + 1.0)
  o_ref[...] = (m + g) * inv + b2_ref[...]


_layer2 = pl.pallas_call(
    _layer2_body,
    grid=(N // BM,),
    in_specs=[
        pl.BlockSpec((NC, BM, HALF), lambda i: (0, i, 0)),
        pl.BlockSpec((NC, BM, HALF), lambda i: (0, i, 0)),
        pl.BlockSpec((NC, BM, HALF), lambda i: (0, i, 0)),
        pl.BlockSpec((1, OUT_F), lambda i: (0, 0)),
    ],
    out_specs=pl.BlockSpec((BM, OUT_F), lambda i: (i, 0)),
    out_shape=jax.ShapeDtypeStruct((N, OUT_F), jnp.float32),
)


def kernel(node_feat, edge_index, W1, b1, W2, b2):
  src = edge_index[0]
  dst = edge_index[1]
  pad = EP - E
  src_p = jnp.concatenate([src, jnp.zeros((pad,), jnp.int32)])
  dst_p = jnp.concatenate([dst, jnp.full((pad,), N, jnp.int32)])
  srcs = jnp.stack([src_p, src_p + N]).reshape(NC, NS, CH, K)
  dsts = dst_p.reshape(NS, CH, K)
  zeros = jnp.zeros((ACC_PT, HALF), jnp.float32)
  ones = jnp.ones((K, HALF), jnp.float32)

  feat_t = _relayout(node_feat)
  degw = _deg(dsts, zeros, ones)
  msg1 = _agg(feat_t.reshape(NC * N, HALF), srcs, dsts, zeros)
  g = _layer1(msg1, node_feat, degw, W1, b1.reshape(1, HID_F), W2)
  msg2 = _agg(g.reshape(NC * N, HALF), srcs, dsts, zeros)
  return _layer2(msg2, g, degw, b2.reshape(1, OUT_F))
